# Initial kernel scaffold; baseline (speedup 1.0000x reference)
import jax
import jax.numpy as jnp
from jax import lax
from jax.experimental import pallas as pl
from jax.experimental.pallas import tpu as pltpu
from jax.experimental.pallas import tpu_sc as plsc

N_NODES = 10000
D = 128
N_EDGES = 320000

NCORES = 2
NSUB = 16
NWORK = NCORES * NSUB
K = 128
CHUNKS = 80
EW = K * CHUNKS
E_PAD = EW * NWORK
NPAD = 10240
ROWS_PER_SUB = NPAD // NSUB
DEGW = 16


def _sc_scatter(x, src_r, dst_r, z128, z16, on16):
    mesh = plsc.VectorSubcoreMesh(core_axis_name="c", subcore_axis_name="s")

    @pl.kernel(
        out_type=[
            jax.ShapeDtypeStruct((NCORES, NPAD, D), jnp.float32),
            jax.ShapeDtypeStruct((NCORES, NPAD, DEGW), jnp.float32),
        ],
        mesh=mesh,
        scratch_types=[
            pltpu.VMEM_SHARED((NPAD, D), jnp.float32),
            pltpu.VMEM_SHARED((NPAD, DEGW), jnp.float32),
            pltpu.VMEM((CHUNKS, K), jnp.int32),
            pltpu.VMEM((CHUNKS, K), jnp.int32),
            pltpu.VMEM((K, D), jnp.float32),
            pltpu.VMEM((K, D), jnp.float32),
            pltpu.VMEM((K, DEGW), jnp.float32),
            pltpu.VMEM((K, DEGW), jnp.float32),
            pltpu.SemaphoreType.DMA,
            pltpu.SemaphoreType.DMA,
        ],
    )
    def k(x_hbm, src_hbm, dst_hbm, z128_hbm, z16_hbm, on16_hbm,
          agg_hbm, dega_hbm,
          acc, dacc, srcv, dstv, rows0, rows1, onesv, z16v, sem0, sem1):
        c = lax.axis_index("c")
        s = lax.axis_index("s")
        w = c * NSUB + s
        row0 = s * ROWS_PER_SUB

        pltpu.sync_copy(z128_hbm, rows0)
        pltpu.sync_copy(z16_hbm, z16v)
        pltpu.sync_copy(on16_hbm, onesv)
        for j in range(ROWS_PER_SUB // K):
            pltpu.sync_copy(rows0, acc.at[pl.ds(row0 + j * K, K)])
            pltpu.sync_copy(z16v, dacc.at[pl.ds(row0 + j * K, K)])
        plsc.subcore_barrier()

        pltpu.sync_copy(src_hbm.at[w], srcv)
        pltpu.sync_copy(dst_hbm.at[w], dstv)

        pltpu.async_copy(x_hbm.at[srcv.at[0]], rows0, sem0)

        @pl.loop(0, CHUNKS, step=2)
        def _(ci):
            pltpu.async_copy(x_hbm.at[srcv.at[ci + 1]], rows1, sem1)
            pltpu.make_async_copy(x_hbm.at[srcv.at[ci]], rows0, sem0).wait()
            pltpu.sync_copy(rows0, acc.at[dstv.at[ci]], add=True)
            pltpu.sync_copy(onesv, dacc.at[dstv.at[ci]], add=True)

            @pl.when(ci + 2 < CHUNKS)
            def _():
                pltpu.async_copy(x_hbm.at[srcv.at[ci + 2]], rows0, sem0)

            pltpu.make_async_copy(x_hbm.at[srcv.at[ci + 1]], rows1, sem1).wait()
            pltpu.sync_copy(rows1, acc.at[dstv.at[ci + 1]], add=True)
            pltpu.sync_copy(onesv, dacc.at[dstv.at[ci + 1]], add=True)

        plsc.subcore_barrier()

        for j in range(ROWS_PER_SUB // K):
            r = row0 + j * K
            pltpu.sync_copy(acc.at[pl.ds(r, K)], agg_hbm.at[c].at[pl.ds(r, K)])
            pltpu.sync_copy(dacc.at[pl.ds(r, K)], dega_hbm.at[c].at[pl.ds(r, K)])

    return k(x, src_r, dst_r, z128, z16, on16)


def _tc_finish(x, a0, a1, d0, d1, wt):
    BLK = 1000

    def body(x_ref, a0_ref, a1_ref, d0_ref, d1_ref, wt_ref, o_ref):
        ssum = x_ref[...] + a0_ref[...] + a1_ref[...]
        m = jnp.dot(ssum, wt_ref[...], preferred_element_type=jnp.float32)
        norm = d0_ref[...][:, :1] + d1_ref[...][:, :1] + 1.0
        o_ref[...] = jnp.maximum(m / norm, 0.0)

    return pl.pallas_call(
        body,
        grid=(N_NODES // BLK,),
        in_specs=[
            pl.BlockSpec((BLK, D), lambda i: (i, 0)),
            pl.BlockSpec((BLK, D), lambda i: (i, 0)),
            pl.BlockSpec((BLK, D), lambda i: (i, 0)),
            pl.BlockSpec((BLK, DEGW), lambda i: (i, 0)),
            pl.BlockSpec((BLK, DEGW), lambda i: (i, 0)),
            pl.BlockSpec((D, D), lambda i: (0, 0)),
        ],
        out_specs=pl.BlockSpec((BLK, D), lambda i: (i, 0)),
        out_shape=jax.ShapeDtypeStruct((N_NODES, D), jnp.float32),
    )(x, a0, a1, d0, d1, wt)


def kernel(x, edge_index, W):
    src = edge_index[0].astype(jnp.int32)
    dst = edge_index[1].astype(jnp.int32)
    npad_e = E_PAD - N_EDGES
    src_p = jnp.concatenate([src, jnp.zeros((npad_e,), jnp.int32)])
    dst_p = jnp.concatenate(
        [dst, N_NODES + (jnp.arange(npad_e, dtype=jnp.int32) % (NPAD - N_NODES))])
    src_r = src_p.reshape(NWORK, CHUNKS, K)
    dst_r = dst_p.reshape(NWORK, CHUNKS, K)

    z128 = jnp.zeros((K, D), jnp.float32)
    z16 = jnp.zeros((K, DEGW), jnp.float32)
    on16 = jnp.ones((K, DEGW), jnp.float32)

    agg, dega = _sc_scatter(x, src_r, dst_r, z128, z16, on16)

    a0 = agg[0, :N_NODES]
    a1 = agg[1, :N_NODES]
    d0 = dega[0, :N_NODES]
    d1 = dega[1, :N_NODES]
    wt = W.T
    return _tc_finish(x, a0, a1, d0, d1, wt)


# speedup vs baseline: 3.0934x; 3.0934x over previous
import jax
import jax.numpy as jnp
from jax import lax
from jax.experimental import pallas as pl
from jax.experimental.pallas import tpu as pltpu
from jax.experimental.pallas import tpu_sc as plsc

N_NODES = 10000
D = 128
N_EDGES = 320000

NCORES = 2
NSUB = 16
NWORK = NCORES * NSUB
EW = N_EDGES // NWORK
K = 80
CHUNKS = 128
REAL_CHUNKS = EW // K
EWP = CHUNKS * K
GC = 8
NG = CHUNKS // GC
ROWS_PER_SUB = 624
TAIL0 = NSUB * ROWS_PER_SUB
TAILN = N_NODES - TAIL0


def _sc_scatter(x, src_r, dst_r, zrow, ones_k):
    mesh = plsc.VectorSubcoreMesh(core_axis_name="c", subcore_axis_name="s")

    @pl.kernel(
        out_type=[
            jax.ShapeDtypeStruct((NCORES, N_NODES, D), jnp.float32),
            jax.ShapeDtypeStruct((NCORES, N_NODES, D), jnp.float32),
        ],
        mesh=mesh,
        scratch_types=[
            pltpu.VMEM_SHARED((N_NODES, D), jnp.float32),
            pltpu.VMEM((GC, K), jnp.int32),
            pltpu.VMEM((GC, K), jnp.int32),
            pltpu.VMEM((K, D), jnp.float32),
            pltpu.VMEM((K, D), jnp.float32),
            pltpu.SemaphoreType.DMA,
            pltpu.SemaphoreType.DMA,
        ],
    )
    def k(x_hbm, src_hbm, dst_hbm, zrow_hbm, ones_hbm,
          agg_hbm, deg_hbm,
          acc, srcv, dstv, rows0, rows1, sem0, sem1):
        c = lax.axis_index("c")
        s = lax.axis_index("s")
        w = c * NSUB + s
        row0 = s * ROWS_PER_SUB

        def zero_acc():
            pltpu.sync_copy(zrow_hbm.at[pl.ds(0, ROWS_PER_SUB)],
                            acc.at[pl.ds(row0, ROWS_PER_SUB)])

            @pl.when(s == 0)
            def _():
                pltpu.sync_copy(zrow_hbm.at[pl.ds(0, TAILN)],
                                acc.at[pl.ds(TAIL0, TAILN)])

        def copy_out(dst_out):
            pltpu.sync_copy(acc.at[pl.ds(row0, ROWS_PER_SUB)],
                            dst_out.at[c].at[pl.ds(row0, ROWS_PER_SUB)])

            @pl.when(s == 0)
            def _():
                pltpu.sync_copy(acc.at[pl.ds(TAIL0, TAILN)],
                                dst_out.at[c].at[pl.ds(TAIL0, TAILN)])

        zero_acc()
        plsc.subcore_barrier()

        for g in range(NG):
            nreal = min(REAL_CHUNKS - g * GC, GC)

            pltpu.sync_copy(src_hbm.at[w].at[pl.ds(g * GC, GC)], srcv)
            pltpu.sync_copy(dst_hbm.at[w].at[pl.ds(g * GC, GC)], dstv)

            pltpu.async_copy(x_hbm.at[srcv.at[0]], rows0, sem0)

            @pl.loop(0, GC, step=2)
            def _(j):
                pltpu.async_copy(x_hbm.at[srcv.at[j + 1]], rows1, sem1)
                pltpu.make_async_copy(x_hbm.at[srcv.at[j]], rows0, sem0).wait()

                if nreal >= GC:
                    pltpu.sync_copy(rows0, acc.at[dstv.at[j]], add=True)
                else:
                    @pl.when(j < nreal)
                    def _():
                        pltpu.sync_copy(rows0, acc.at[dstv.at[j]], add=True)

                @pl.when(j + 2 < GC)
                def _():
                    pltpu.async_copy(x_hbm.at[srcv.at[j + 2]], rows0, sem0)

                pltpu.make_async_copy(x_hbm.at[srcv.at[j + 1]], rows1, sem1).wait()

                if nreal >= GC:
                    pltpu.sync_copy(rows1, acc.at[dstv.at[j + 1]], add=True)
                else:
                    @pl.when(j + 1 < nreal)
                    def _():
                        pltpu.sync_copy(rows1, acc.at[dstv.at[j + 1]], add=True)

        plsc.subcore_barrier()
        copy_out(agg_hbm)
        plsc.subcore_barrier()

        zero_acc()
        pltpu.sync_copy(ones_hbm, rows0)
        plsc.subcore_barrier()

        for g in range(NG):
            nreal = min(REAL_CHUNKS - g * GC, GC)
            pltpu.sync_copy(dst_hbm.at[w].at[pl.ds(g * GC, GC)], dstv)

            @pl.loop(0, GC)
            def _(j):
                if nreal >= GC:
                    pltpu.sync_copy(rows0, acc.at[dstv.at[j]], add=True)
                else:
                    @pl.when(j < nreal)
                    def _():
                        pltpu.sync_copy(rows0, acc.at[dstv.at[j]], add=True)

        plsc.subcore_barrier()
        copy_out(deg_hbm)

    return k(x, src_r, dst_r, zrow, ones_k)


def _tc_finish(x, a0, a1, d0, d1, wt):
    BLK = 1000

    def body(x_ref, a0_ref, a1_ref, d0_ref, d1_ref, wt_ref, o_ref):
        ssum = x_ref[...] + a0_ref[...] + a1_ref[...]
        m = jnp.dot(ssum, wt_ref[...], preferred_element_type=jnp.float32)
        norm = d0_ref[...][:, :1] + d1_ref[...][:, :1] + 1.0
        o_ref[...] = jnp.maximum(m / norm, 0.0)

    return pl.pallas_call(
        body,
        grid=(N_NODES // BLK,),
        in_specs=[
            pl.BlockSpec((BLK, D), lambda i: (i, 0)),
            pl.BlockSpec((BLK, D), lambda i: (i, 0)),
            pl.BlockSpec((BLK, D), lambda i: (i, 0)),
            pl.BlockSpec((BLK, D), lambda i: (i, 0)),
            pl.BlockSpec((BLK, D), lambda i: (i, 0)),
            pl.BlockSpec((D, D), lambda i: (0, 0)),
        ],
        out_specs=pl.BlockSpec((BLK, D), lambda i: (i, 0)),
        out_shape=jax.ShapeDtypeStruct((N_NODES, D), jnp.float32),
    )(x, a0, a1, d0, d1, wt)


def kernel(x, edge_index, W):
    src = edge_index[0].astype(jnp.int32)
    dst = edge_index[1].astype(jnp.int32)
    src_r = jnp.pad(src.reshape(NWORK, EW), ((0, 0), (0, EWP - EW)))
    dst_r = jnp.pad(dst.reshape(NWORK, EW), ((0, 0), (0, EWP - EW)))
    src_r = src_r.reshape(NWORK, CHUNKS, K)
    dst_r = dst_r.reshape(NWORK, CHUNKS, K)

    zrow = jnp.zeros((ROWS_PER_SUB, D), jnp.float32)
    ones_k = jnp.ones((K, D), jnp.float32)

    agg, degt = _sc_scatter(x, src_r, dst_r, zrow, ones_k)

    return _tc_finish(x, agg[0], agg[1], degt[0], degt[1], W.T)


# speedup vs baseline: 3.5767x; 1.1562x over previous
import dataclasses

import jax
import jax.numpy as jnp
from jax import lax
from jax.experimental import pallas as pl
from jax.experimental.pallas import tpu as pltpu
from jax.experimental.pallas import tpu_sc as plsc

N_NODES = 10000
D = 128
N_EDGES = 320000

NCORES = 2
NSUB = 16
NWORK = NCORES * NSUB
EW = N_EDGES // NWORK
K = 80
CHUNKS = 128
REAL_CHUNKS = EW // K
EWP = CHUNKS * K
GC = 8
NG = CHUNKS // GC
ROWS_PER_SUB = 624
TAIL0 = NSUB * ROWS_PER_SUB
TAILN = N_NODES - TAIL0
VL = 16


def _sc_scatter(x, src_r, dst_r, zrow, zhist):
    mesh = plsc.VectorSubcoreMesh(core_axis_name="c", subcore_axis_name="s")
    cp = pltpu.CompilerParams()
    if "needs_layout_passes" in pltpu.CompilerParams.__dataclass_fields__:
        cp = dataclasses.replace(cp, needs_layout_passes=False)

    @pl.kernel(
        compiler_params=cp,
        out_type=[
            jax.ShapeDtypeStruct((NCORES, N_NODES, D), jnp.float32),
            jax.ShapeDtypeStruct((NCORES, NSUB, N_NODES), jnp.float32),
        ],
        mesh=mesh,
        scratch_types=[
            pltpu.VMEM_SHARED((N_NODES, D), jnp.float32),
            pltpu.VMEM((GC, K), jnp.int32),
            pltpu.VMEM((GC, K), jnp.int32),
            pltpu.VMEM((K, D), jnp.float32),
            pltpu.VMEM((K, D), jnp.float32),
            pltpu.VMEM((N_NODES,), jnp.float32),
            pltpu.SemaphoreType.DMA,
            pltpu.SemaphoreType.DMA,
        ],
    )
    def k(x_hbm, src_hbm, dst_hbm, zrow_hbm, zhist_hbm,
          agg_hbm, hist_hbm,
          acc, srcv, dstv, rows0, rows1, hist, sem0, sem1):
        c = lax.axis_index("c")
        s = lax.axis_index("s")
        w = c * NSUB + s
        row0 = s * ROWS_PER_SUB
        ones_v = jnp.full((VL,), 1.0, jnp.float32)

        pltpu.sync_copy(zrow_hbm.at[pl.ds(0, ROWS_PER_SUB)],
                        acc.at[pl.ds(row0, ROWS_PER_SUB)])
        pltpu.sync_copy(zhist_hbm, hist)

        @pl.when(s == 0)
        def _():
            pltpu.sync_copy(zrow_hbm.at[pl.ds(0, TAILN)],
                            acc.at[pl.ds(TAIL0, TAILN)])

        plsc.subcore_barrier()

        def hist_chunk(j):
            for l in range(0, K, VL):
                idxv = dstv[j, pl.ds(l, VL)]
                plsc.addupdate_scatter(hist, [idxv], ones_v)

        for g in range(NG):
            nreal = min(REAL_CHUNKS - g * GC, GC)

            pltpu.sync_copy(src_hbm.at[w].at[pl.ds(g * GC, GC)], srcv)
            pltpu.sync_copy(dst_hbm.at[w].at[pl.ds(g * GC, GC)], dstv)

            pltpu.async_copy(x_hbm.at[srcv.at[0]], rows0, sem0)

            @pl.loop(0, GC, step=2)
            def _(j):
                pltpu.async_copy(x_hbm.at[srcv.at[j + 1]], rows1, sem1)
                pltpu.make_async_copy(x_hbm.at[srcv.at[j]], rows0, sem0).wait()

                if nreal >= GC:
                    pltpu.sync_copy(rows0, acc.at[dstv.at[j]], add=True)
                    hist_chunk(j)
                else:
                    @pl.when(j < nreal)
                    def _():
                        pltpu.sync_copy(rows0, acc.at[dstv.at[j]], add=True)
                        hist_chunk(j)

                @pl.when(j + 2 < GC)
                def _():
                    pltpu.async_copy(x_hbm.at[srcv.at[j + 2]], rows0, sem0)

                pltpu.make_async_copy(x_hbm.at[srcv.at[j + 1]], rows1, sem1).wait()

                if nreal >= GC:
                    pltpu.sync_copy(rows1, acc.at[dstv.at[j + 1]], add=True)
                    hist_chunk(j + 1)
                else:
                    @pl.when(j + 1 < nreal)
                    def _():
                        pltpu.sync_copy(rows1, acc.at[dstv.at[j + 1]], add=True)
                        hist_chunk(j + 1)

        plsc.subcore_barrier()

        pltpu.sync_copy(acc.at[pl.ds(row0, ROWS_PER_SUB)],
                        agg_hbm.at[c].at[pl.ds(row0, ROWS_PER_SUB)])
        pltpu.sync_copy(hist, hist_hbm.at[c].at[s])

        @pl.when(s == 0)
        def _():
            pltpu.sync_copy(acc.at[pl.ds(TAIL0, TAILN)],
                            agg_hbm.at[c].at[pl.ds(TAIL0, TAILN)])

    return k(x, src_r, dst_r, zrow, zhist)


def _tc_finish(x, a0, a1, ht, wt):
    BLK = 1000

    def body(x_ref, a0_ref, a1_ref, h_ref, wt_ref, o_ref):
        ssum = x_ref[...] + a0_ref[...] + a1_ref[...]
        m = jnp.dot(ssum, wt_ref[...], preferred_element_type=jnp.float32)
        norm = jnp.sum(h_ref[...], axis=1, keepdims=True) + 1.0
        o_ref[...] = jnp.maximum(m / norm, 0.0)

    return pl.pallas_call(
        body,
        grid=(N_NODES // BLK,),
        in_specs=[
            pl.BlockSpec((BLK, D), lambda i: (i, 0)),
            pl.BlockSpec((BLK, D), lambda i: (i, 0)),
            pl.BlockSpec((BLK, D), lambda i: (i, 0)),
            pl.BlockSpec((BLK, NWORK), lambda i: (i, 0)),
            pl.BlockSpec((D, D), lambda i: (0, 0)),
        ],
        out_specs=pl.BlockSpec((BLK, D), lambda i: (i, 0)),
        out_shape=jax.ShapeDtypeStruct((N_NODES, D), jnp.float32),
    )(x, a0, a1, ht, wt)


def kernel(x, edge_index, W):
    src = edge_index[0].astype(jnp.int32)
    dst = edge_index[1].astype(jnp.int32)
    src_r = jnp.pad(src.reshape(NWORK, EW), ((0, 0), (0, EWP - EW)))
    dst_r = jnp.pad(dst.reshape(NWORK, EW), ((0, 0), (0, EWP - EW)))
    src_r = src_r.reshape(NWORK, CHUNKS, K)
    dst_r = dst_r.reshape(NWORK, CHUNKS, K)

    zrow = jnp.zeros((ROWS_PER_SUB, D), jnp.float32)
    zhist = jnp.zeros((N_NODES,), jnp.float32)

    agg, hist = _sc_scatter(x, src_r, dst_r, zrow, zhist)

    ht = hist.reshape(NWORK, N_NODES).T
    return _tc_finish(x, agg[0], agg[1], ht, W.T)
